# baseline (speedup 1.0000x reference)
import functools

import jax
import jax.numpy as jnp
from jax import lax
from jax.experimental import pallas as pl
from jax.experimental.pallas import tpu as pltpu
from jax.experimental.pallas import tpu_sc as plsc

NUM_PARTITIONS = 8
P = NUM_PARTITIONS
NW = 32
CHUNK = 16
TSC = 2560
TBLK = 256


def _sc_body(Tw, K, D, x_hbm, idx_hbm, out_hbm,
             xbuf, idxbuf, zrow, xsem, wsem):
    nchunks = Tw // CHUNK
    wid = lax.axis_index("s") * 2 + lax.axis_index("c")
    tbase = wid * Tw

    for v in range(D // 16):
        zrow[0, pl.ds(v * 16, 16)] = jnp.zeros((16,), jnp.float32)

    pltpu.sync_copy(idx_hbm.at[pl.ds(tbase * K, Tw * K)],
                    idxbuf.at[pl.ds(0, Tw * K)])

    def load_chunk(ci):
        pltpu.make_async_copy(
            x_hbm.at[pl.ds(tbase + ci * CHUNK, CHUNK)],
            xbuf.at[ci % 2], xsem).start()

    def wait_chunk(ci):
        pltpu.make_async_copy(
            x_hbm.at[pl.ds(tbase + ci * CHUNK, CHUNK)],
            xbuf.at[ci % 2], xsem).wait()

    def drain_rows(n):
        def body(j, carry):
            pltpu.make_async_copy(
                x_hbm.at[pl.ds(0, 1)], zrow, wsem).wait()
            return carry
        lax.fori_loop(0, n, body, 0)

    load_chunk(0)
    for ci in range(nchunks):
        wait_chunk(ci)
        slot = ci % 2

        def tok_body(t, carry, ci=ci, slot=slot):
            tloc = ci * CHUNK + t
            g = tbase + tloc
            iv = idxbuf[pl.ds(tloc * K, 16)]
            i0 = iv[0]
            i1 = iv[1] if K > 1 else i0
            for p in range(NUM_PARTITIONS):
                sel = (i0 == p) | (i1 == p)
                dst = out_hbm.at[pl.ds(g * P + p, 1)]

                @pl.when(sel)
                def _():
                    pltpu.make_async_copy(
                        xbuf.at[slot, pl.ds(t, 1)], dst, wsem).start()

                @pl.when(jnp.logical_not(sel))
                def _():
                    pltpu.make_async_copy(zrow, dst, wsem).start()
            return carry

        lax.fori_loop(0, CHUNK, tok_body, 0)

        if ci + 1 < nchunks:
            if ci >= 1:
                drain_rows(CHUNK * P)
            load_chunk(ci + 1)
    drain_rows(min(2, nchunks) * CHUNK * P)


def _sc_part(x2d, idxf, Tsc, K, D):
    Tw = Tsc // NW
    body = functools.partial(_sc_body, Tw, K, D)
    out = pl.kernel(
        body,
        out_type=jax.ShapeDtypeStruct((Tsc * P, D), jnp.float32),
        mesh=plsc.VectorSubcoreMesh(core_axis_name="c", subcore_axis_name="s"),
        scratch_types=[
            pltpu.VMEM((2, CHUNK, D), jnp.float32),
            pltpu.VMEM((Tw * K + 16,), jnp.int32),
            pltpu.VMEM((1, D), jnp.float32),
            pltpu.SemaphoreType.DMA,
            pltpu.SemaphoreType.DMA,
        ],
    )(x2d, idxf)
    return out.reshape(Tsc, P, D)


def _tc_kernel(idx_ref, x_ref, out_ref):
    K = idx_ref.shape[1]
    piota = jax.lax.broadcasted_iota(
        jnp.int32, (TBLK, NUM_PARTITIONS, 1), 1)
    m = idx_ref[:, 0:1, :] == piota
    for k in range(1, K):
        m = m | (idx_ref[:, k:k + 1, :] == piota)
    out_ref[...] = jnp.where(m, x_ref[...], 0.0)


def _tc_part(x3d, idx3d, t0, Ttc, D, K):
    off = t0 // TBLK
    out = pl.pallas_call(
        _tc_kernel,
        grid=(Ttc // TBLK,),
        in_specs=[
            pl.BlockSpec((TBLK, K, 1), lambda i: (i + off, 0, 0)),
            pl.BlockSpec((TBLK, 1, D), lambda i: (i + off, 0, 0)),
        ],
        out_specs=pl.BlockSpec((TBLK, NUM_PARTITIONS, D),
                               lambda i: (i, 0, 0)),
        out_shape=jax.ShapeDtypeStruct((Ttc, NUM_PARTITIONS, D),
                                       jnp.float32),
    )(idx3d, x3d)
    return out


def kernel(x, partition_indices):
    B, S, D = x.shape
    T = B * S
    K = partition_indices.shape[-1]
    x2d = x.reshape(T, D)
    x3d = x.reshape(T, 1, D)
    idxf = partition_indices.reshape(T * K).astype(jnp.int32)
    idx3d = partition_indices.reshape(T, K, 1).astype(jnp.int32)

    sc_out = _sc_part(x2d, idxf, TSC, K, D)
    tc_out = _tc_part(x3d, idx3d, TSC, T - TSC, D, K)
    out = jnp.concatenate([sc_out, tc_out], axis=0)
    return out.reshape(B, S, P, D)


# speedup vs baseline: 2.4756x; 2.4756x over previous
import functools

import jax
import jax.numpy as jnp
from jax import lax
from jax.experimental import pallas as pl
from jax.experimental.pallas import tpu as pltpu
from jax.experimental.pallas import tpu_sc as plsc

NUM_PARTITIONS = 8
P = NUM_PARTITIONS
NW = 32
CHUNK = 16


def _sc_body(Tw, K, D, x_hbm, idx_hbm, out_hbm,
             xbuf, idxbuf, zrow, xsem, wsem):
    nchunks = Tw // CHUNK
    wid = lax.axis_index("s") * 2 + lax.axis_index("c")
    tbase = wid * Tw

    for v in range(D // 16):
        zrow[0, pl.ds(v * 16, 16)] = jnp.zeros((16,), jnp.float32)

    pltpu.sync_copy(idx_hbm.at[pl.ds(tbase * K, Tw * K)],
                    idxbuf.at[pl.ds(0, Tw * K)])

    def load_chunk(ci):
        pltpu.make_async_copy(
            x_hbm.at[pl.ds(tbase + ci * CHUNK, CHUNK)],
            xbuf.at[ci % 2],
            xsem,
        ).start()

    def wait_chunk(ci):
        pltpu.make_async_copy(
            x_hbm.at[pl.ds(tbase + ci * CHUNK, CHUNK)],
            xbuf.at[ci % 2],
            xsem,
        ).wait()

    def drain_rows(n):
        def body(j, carry):
            pltpu.make_async_copy(
                x_hbm.at[pl.ds(0, 1)], zrow, wsem).wait()
            return carry
        lax.fori_loop(0, n, body, 0)

    load_chunk(0)
    for ci in range(nchunks):
        wait_chunk(ci)
        slot = ci % 2

        def tok_body(t, carry, ci=ci, slot=slot):
            tloc = ci * CHUNK + t
            g = tbase + tloc
            iv = idxbuf[pl.ds(tloc * K, 16)]
            i0 = iv[0]
            i1 = iv[1] if K > 1 else i0
            for p in range(NUM_PARTITIONS):
                sel = (i0 == p) | (i1 == p)
                dst = out_hbm.at[pl.ds(g * P + p, 1)]

                @pl.when(sel)
                def _():
                    pltpu.make_async_copy(
                        xbuf.at[slot, pl.ds(t, 1)], dst, wsem).start()

                @pl.when(jnp.logical_not(sel))
                def _():
                    pltpu.make_async_copy(zrow, dst, wsem).start()
            return carry

        lax.fori_loop(0, CHUNK, tok_body, 0)

        if ci + 1 < nchunks:
            if ci >= 1:
                drain_rows(CHUNK * P)
            load_chunk(ci + 1)
    drain_rows(min(2, nchunks) * CHUNK * P)


def kernel(x, partition_indices):
    B, S, D = x.shape
    T = B * S
    K = partition_indices.shape[-1]
    Tw = T // NW
    x2d = x.reshape(T, D)
    idxf = partition_indices.reshape(T * K).astype(jnp.int32)

    body = functools.partial(_sc_body, Tw, K, D)
    out = pl.kernel(
        body,
        out_type=jax.ShapeDtypeStruct((T * P, D), jnp.float32),
        mesh=plsc.VectorSubcoreMesh(core_axis_name="c", subcore_axis_name="s"),
        scratch_types=[
            pltpu.VMEM((2, CHUNK, D), jnp.float32),
            pltpu.VMEM((Tw * K + 16,), jnp.int32),
            pltpu.VMEM((1, D), jnp.float32),
            pltpu.SemaphoreType.DMA,
            pltpu.SemaphoreType.DMA,
        ],
    )(x2d, idxf)
    return out.reshape(B, S, P, D)
